# Initial kernel scaffold; baseline (speedup 1.0000x reference)
import math

import jax
import jax.numpy as jnp
from jax.experimental import pallas as pl

B = 1024
C = 100000
M = 0.4
S = 64.0
EPS = 1e-3
COS_EPS = math.cos(EPS)
COL_BLK = 2000


def _main_kernel(norms_ref, labels_ref, logits_ref, out_ref):
    j = pl.program_id(0)

    safe = jnp.clip(norms_ref[...], 1e-3, 100.0)
    mean = jnp.mean(safe)
    var = jnp.sum((safe - mean) ** 2) / (B - 1)
    std = jnp.sqrt(var)
    ms = (safe - mean) / (std + EPS)
    g = -M * ms
    cos_g = jnp.cos(g)
    sin_g = jnp.sin(g)
    thr_hi = jnp.cos(jnp.clip(EPS - g, 0.0, math.pi))
    thr_lo = jnp.cos(jnp.clip(math.pi - EPS - g, 0.0, math.pi))
    g_add = M + M * ms

    x = logits_ref[...]
    col = j * COL_BLK + jax.lax.broadcasted_iota(jnp.int32, x.shape, 1)
    is_target = col == labels_ref[...]

    dense = jnp.clip(x, -COS_EPS, COS_EPS)

    spec = x * cos_g - jnp.sqrt(jnp.maximum(1.0 - x * x, 0.0)) * sin_g
    spec = jnp.where(x > thr_hi, COS_EPS, spec)
    spec = jnp.where(x < thr_lo, -COS_EPS, spec)
    spec = spec - g_add

    out_ref[...] = S * jnp.where(is_target, spec, dense)


@jax.jit
def kernel(logits, norms, labels):
    labels2d = labels.astype(jnp.int32).reshape(B, 1)
    grid = (C + COL_BLK - 1) // COL_BLK
    return pl.pallas_call(
        _main_kernel,
        grid=(grid,),
        in_specs=[
            pl.BlockSpec((B, 1), lambda j: (0, 0)),
            pl.BlockSpec((B, 1), lambda j: (0, 0)),
            pl.BlockSpec((B, COL_BLK), lambda j: (0, j)),
        ],
        out_specs=pl.BlockSpec((B, COL_BLK), lambda j: (0, j)),
        out_shape=jax.ShapeDtypeStruct((B, C), logits.dtype),
    )(norms, labels2d, logits)


# speedup vs baseline: 5.5175x; 5.5175x over previous
import math

import jax
import jax.numpy as jnp
from jax.experimental import pallas as pl

B = 1024
C = 100000
M = 0.4
S = 64.0
EPS = 1e-3
COS_EPS = math.cos(EPS)
COL_BLK = 2048


def _main_kernel(norms_ref, labels_ref, logits_ref, out_ref):
    j = pl.program_id(0)

    safe = jnp.clip(norms_ref[...], 1e-3, 100.0)
    mean = jnp.mean(safe)
    var = jnp.sum((safe - mean) ** 2) / (B - 1)
    std = jnp.sqrt(var)
    ms = (safe - mean) / (std + EPS)
    g = -M * ms
    cos_g = jnp.cos(g)
    sin_g = jnp.sin(g)
    thr_hi = jnp.cos(jnp.clip(EPS - g, 0.0, math.pi))
    thr_lo = jnp.cos(jnp.clip(math.pi - EPS - g, 0.0, math.pi))
    g_add = M + M * ms

    x = logits_ref[...]
    col = j * COL_BLK + jax.lax.broadcasted_iota(jnp.int32, x.shape, 1)
    is_target = col == labels_ref[...]

    dense = jnp.clip(x, -COS_EPS, COS_EPS)

    spec = x * cos_g - jnp.sqrt(jnp.maximum(1.0 - x * x, 0.0)) * sin_g
    spec = jnp.where(x > thr_hi, COS_EPS, spec)
    spec = jnp.where(x < thr_lo, -COS_EPS, spec)
    spec = spec - g_add

    out_ref[...] = S * jnp.where(is_target, spec, dense)


@jax.jit
def kernel(logits, norms, labels):
    labels2d = labels.astype(jnp.int32).reshape(B, 1)
    grid = (C + COL_BLK - 1) // COL_BLK
    return pl.pallas_call(
        _main_kernel,
        grid=(grid,),
        in_specs=[
            pl.BlockSpec((B, 1), lambda j: (0, 0)),
            pl.BlockSpec((B, 1), lambda j: (0, 0)),
            pl.BlockSpec((B, COL_BLK), lambda j: (0, j)),
        ],
        out_specs=pl.BlockSpec((B, COL_BLK), lambda j: (0, j)),
        out_shape=jax.ShapeDtypeStruct((B, C), logits.dtype),
    )(norms, labels2d, logits)


# speedup vs baseline: 6.9469x; 1.2591x over previous
import math

import jax
import jax.numpy as jnp
from jax.experimental import pallas as pl
from jax.experimental.pallas import tpu as pltpu

B = 1024
C = 100000
M = 0.4
S = 64.0
EPS = 1e-3
COS_EPS = math.cos(EPS)
COL_BLK = 2048


def _main_kernel(norms_ref, labels_ref, logits_ref, out_ref,
                 cos_g_s, sin_g_s, thr_hi_s, thr_lo_s, g_add_s):
    j = pl.program_id(0)

    @pl.when(j == 0)
    def _prologue():
        safe = jnp.clip(norms_ref[...], 1e-3, 100.0)
        mean = jnp.mean(safe)
        var = jnp.sum((safe - mean) ** 2) / (B - 1)
        std = jnp.sqrt(var)
        ms = (safe - mean) / (std + EPS)
        g = -M * ms
        cos_g_s[...] = jnp.cos(g)
        sin_g_s[...] = jnp.sin(g)
        thr_hi_s[...] = jnp.cos(jnp.clip(EPS - g, 0.0, math.pi))
        thr_lo_s[...] = jnp.cos(jnp.clip(math.pi - EPS - g, 0.0, math.pi))
        g_add_s[...] = M + M * ms

    x = logits_ref[...]
    lab_local = labels_ref[...] - j * COL_BLK
    col = jax.lax.broadcasted_iota(jnp.int32, x.shape, 1)
    is_target = col == lab_local

    dense = jnp.clip(x, -COS_EPS, COS_EPS)

    spec = x * cos_g_s[...] - jnp.sqrt(jnp.maximum(1.0 - x * x, 0.0)) * sin_g_s[...]
    spec = jnp.where(x > thr_hi_s[...], COS_EPS, spec)
    spec = jnp.where(x < thr_lo_s[...], -COS_EPS, spec)
    spec = spec - g_add_s[...]

    out_ref[...] = S * jnp.where(is_target, spec, dense)


@jax.jit
def kernel(logits, norms, labels):
    labels2d = labels.astype(jnp.int32).reshape(B, 1)
    grid = (C + COL_BLK - 1) // COL_BLK
    return pl.pallas_call(
        _main_kernel,
        grid=(grid,),
        in_specs=[
            pl.BlockSpec((B, 1), lambda j: (0, 0)),
            pl.BlockSpec((B, 1), lambda j: (0, 0)),
            pl.BlockSpec((B, COL_BLK), lambda j: (0, j)),
        ],
        out_specs=pl.BlockSpec((B, COL_BLK), lambda j: (0, j)),
        out_shape=jax.ShapeDtypeStruct((B, C), logits.dtype),
        scratch_shapes=[pltpu.VMEM((B, 1), jnp.float32)] * 5,
    )(norms, labels2d, logits)
